# Initial kernel scaffold; baseline (speedup 1.0000x reference)
import jax
import jax.numpy as jnp
from jax.experimental import pallas as pl


def kernel(poss_edge, edge_src, edge_weight, neighbours_sum):
    raise NotImplementedError("write your pallas kernel here")


# speedup vs baseline: 3.3786x; 3.3786x over previous
import functools

import jax
import jax.numpy as jnp
from jax import lax
from jax.experimental import pallas as pl
from jax.experimental.pallas import tpu as pltpu
from jax.experimental.pallas import tpu_sc as plsc

N_NODES = 100000
N_EDGES = 3200000
D = 11

NC = 2
NS = 16
NW = NC * NS
NPT = N_NODES // NW
C = 2000
NCHUNK = N_EDGES // C
NG = C // 16
ACC_W = ((NPT * D + 15) // 16) * 16
NB_W = ((NPT + 7 + 15) // 16) * 16


def _build(interpret=False):
  mesh = plsc.VectorSubcoreMesh(
      core_axis_name="c", subcore_axis_name="s",
      num_cores=NC, num_subcores=NS)

  @functools.partial(
      pl.kernel,
      out_type=jax.ShapeDtypeStruct((NW, ACC_W), jnp.float32),
      mesh=mesh,
      scratch_types=[
          pltpu.VMEM((ACC_W,), jnp.float32),
          pltpu.VMEM((C * D,), jnp.float32),
          pltpu.VMEM((C,), jnp.int32),
          pltpu.VMEM((C,), jnp.float32),
          pltpu.VMEM((NB_W,), jnp.float32),
          pltpu.VMEM((NCHUNK,), jnp.int32),
          pltpu.VMEM((NCHUNK,), jnp.int32),
          pltpu.VMEM((16,), jnp.int32),
          pltpu.SemaphoreType.DMA,
      ],
      compiler_params=pltpu.CompilerParams(needs_layout_passes=False),
      interpret=interpret,
  )
  def seg_kernel(rows_hbm, src_hbm, w_hbm, nb_hbm, samp0_hbm, samp1_hbm,
                 out_hbm, acc_v, rows_v, src_v, w_v, nbr_v, s0_v, s1_v,
                 lane_v, sem):
    wid = lax.axis_index("c") * NS + lax.axis_index("s")
    base = wid * NPT
    limit = base + NPT

    s8 = base - lax.rem(base, 8)
    s8 = pl.multiple_of(jnp.minimum(s8, N_NODES - NB_W), 8)
    off = base - s8
    c1 = pltpu.async_copy(samp0_hbm, s0_v, sem)
    c2 = pltpu.async_copy(samp1_hbm, s1_v, sem)
    c3 = pltpu.async_copy(nb_hbm.at[pl.ds(s8, NB_W)], nbr_v, sem)
    c3.wait()
    c2.wait()
    c1.wait()

    zeros16f = jnp.zeros((16,), jnp.float32)
    iota16 = lax.iota(jnp.int32, 16)

    def init_body(i, _):
      acc_v[pl.ds(i * 16, 16)] = zeros16f
      return 0
    lax.fori_loop(0, ACC_W // 16, init_body, 0)

    def rcp_body(i, _):
      nbr_v[pl.ds(i * 16, 16)] = 1.0 / nbr_v[pl.ds(i * 16, 16)]
      return 0
    lax.fori_loop(0, NB_W // 16, rcp_body, 0)

    def cnt_body(m, carry):
      ca, cb = carry
      va = s1_v[pl.ds(m * 16, 16)]
      vb = s0_v[pl.ds(m * 16, 16)]
      ca = ca + jnp.where(va < base, 1, 0)
      cb = cb + jnp.where(vb < limit, 1, 0)
      return ca, cb
    cnt0, cnt1 = lax.fori_loop(
        0, NCHUNK // 16, cnt_body,
        (jnp.zeros((16,), jnp.int32), jnp.zeros((16,), jnp.int32)))

    def lane_sum(v):
      s = v[0]
      for l in range(1, 16):
        s = s + v[l]
      return s

    k0 = lane_sum(cnt0)
    k1 = lane_sum(cnt1)

    colbase = iota16 * D

    def chunk_body(k, _):
      ro = pl.multiple_of(k * (C * D), 8)
      eo = pl.multiple_of(k * C, 8)
      d1 = pltpu.async_copy(rows_hbm.at[pl.ds(ro, C * D)], rows_v, sem)
      d2 = pltpu.async_copy(src_hbm.at[pl.ds(eo, C)], src_v, sem)
      d3 = pltpu.async_copy(w_hbm.at[pl.ds(eo, C)], w_v, sem)
      d3.wait()
      d2.wait()
      d1.wait()

      def group_body(g, _):
        s16 = src_v[pl.ds(g * 16, 16)]
        w16 = w_v[pl.ds(g * 16, 16)]
        j = s16 - base
        inr = (j >= 0) & (j < NPT)
        jc = jnp.where(inr, j, 0)
        rcp16 = plsc.load_gather(nbr_v, [jc + off])
        wz = jnp.where(inr, w16, 0.0) * rcp16
        j11 = jc * D
        rb = colbase + g * (16 * D)
        for c in range(D):
          col = plsc.load_gather(rows_v, [rb + c])
          plsc.addupdate_scatter(acc_v, [j11 + c], col * wz)
        return 0
      lax.fori_loop(0, NG, group_body, 0)
      return 0

    lax.fori_loop(k0, k1, chunk_body, 0)

    pltpu.sync_copy(acc_v, out_hbm.at[wid])

  return seg_kernel


_seg_kernel = _build()


def kernel(poss_edge, edge_src, edge_weight, neighbours_sum):
  rows_flat = poss_edge.reshape(-1)
  samp0 = edge_src[::C]
  samp1 = jnp.concatenate([samp0[1:], edge_src[-1:]])
  out_raw = _seg_kernel(rows_flat, edge_src, edge_weight, neighbours_sum,
                        samp0, samp1)
  return out_raw[:, : NPT * D].reshape(N_NODES, D)


# speedup vs baseline: 3.4639x; 1.0252x over previous
import functools

import jax
import jax.numpy as jnp
from jax import lax
from jax.experimental import pallas as pl
from jax.experimental.pallas import tpu as pltpu
from jax.experimental.pallas import tpu_sc as plsc

N_NODES = 100000
N_EDGES = 3200000
D = 11

NC = 2
NS = 16
NW = NC * NS
NPT = N_NODES // NW
C = 2000
NCHUNK = N_EDGES // C
NG = C // 16
ACC_W = ((NPT * D + 15) // 16) * 16
NB_W = ((NPT + 7 + 15) // 16) * 16


def _build(interpret=False):
  mesh = plsc.VectorSubcoreMesh(
      core_axis_name="c", subcore_axis_name="s",
      num_cores=NC, num_subcores=NS)

  @functools.partial(
      pl.kernel,
      out_type=jax.ShapeDtypeStruct((NW, ACC_W), jnp.float32),
      mesh=mesh,
      scratch_types=[
          pltpu.VMEM((ACC_W,), jnp.float32),
          pltpu.VMEM((2 * C * D,), jnp.float32),
          pltpu.VMEM((2 * C,), jnp.int32),
          pltpu.VMEM((2 * C,), jnp.float32),
          pltpu.VMEM((NB_W,), jnp.float32),
          pltpu.VMEM((NCHUNK,), jnp.int32),
          pltpu.VMEM((NCHUNK,), jnp.int32),
          pltpu.SemaphoreType.DMA,
          pltpu.SemaphoreType.DMA,
          pltpu.SemaphoreType.DMA,
      ],
      compiler_params=pltpu.CompilerParams(needs_layout_passes=False),
      interpret=interpret,
  )
  def seg_kernel(rows_hbm, src_hbm, w_hbm, nb_hbm, lb_hbm, ub_hbm,
                 out_hbm, acc_v, rows_v, src_v, w_v, nbr_v, lb_v, ub_v,
                 sem_r, sem_s, sem_w):
    wid = lax.axis_index("c") * NS + lax.axis_index("s")
    base = wid * NPT
    limit = base + NPT

    s8 = base - lax.rem(base, 8)
    s8 = pl.multiple_of(jnp.minimum(s8, N_NODES - NB_W), 8)
    off = base - s8
    c1 = pltpu.async_copy(lb_hbm, lb_v, sem_r)
    c2 = pltpu.async_copy(ub_hbm, ub_v, sem_s)
    c3 = pltpu.async_copy(nb_hbm.at[pl.ds(s8, NB_W)], nbr_v, sem_w)
    c3.wait()
    c2.wait()
    c1.wait()

    zeros16f = jnp.zeros((16,), jnp.float32)
    iota16 = lax.iota(jnp.int32, 16)

    def init_body(i, _):
      acc_v[pl.ds(i * 16, 16)] = zeros16f
      return 0
    lax.fori_loop(0, ACC_W // 16, init_body, 0)

    def rcp_body(i, _):
      nbr_v[pl.ds(i * 16, 16)] = 1.0 / nbr_v[pl.ds(i * 16, 16)]
      return 0
    lax.fori_loop(0, NB_W // 16, rcp_body, 0)

    def cnt_body(m, carry):
      ca, cb = carry
      va = ub_v[pl.ds(m * 16, 16)]
      vb = lb_v[pl.ds(m * 16, 16)]
      ca = ca + jnp.where(va < base, 1, 0)
      cb = cb + jnp.where(vb < limit, 1, 0)
      return ca, cb
    cnt0, cnt1 = lax.fori_loop(
        0, NCHUNK // 16, cnt_body,
        (jnp.zeros((16,), jnp.int32), jnp.zeros((16,), jnp.int32)))

    def lane_sum(v):
      s = v[0]
      for l in range(1, 16):
        s = s + v[l]
      return s

    k0 = lane_sum(cnt0)
    k1 = lane_sum(cnt1)

    colbase = iota16 * D

    def issue(k, p):
      ro = pl.multiple_of(k * (C * D), 8)
      eo = pl.multiple_of(k * C, 8)
      po_r = pl.multiple_of(p * (C * D), 8)
      po_e = pl.multiple_of(p * C, 8)
      pltpu.async_copy(rows_hbm.at[pl.ds(ro, C * D)],
                       rows_v.at[pl.ds(po_r, C * D)], sem_r)
      pltpu.async_copy(src_hbm.at[pl.ds(eo, C)],
                       src_v.at[pl.ds(po_e, C)], sem_s)
      pltpu.async_copy(w_hbm.at[pl.ds(eo, C)],
                       w_v.at[pl.ds(po_e, C)], sem_w)

    def drain(p):
      po_r = pl.multiple_of(p * (C * D), 8)
      po_e = pl.multiple_of(p * C, 8)
      pltpu.make_async_copy(rows_hbm.at[pl.ds(0, C * D)],
                            rows_v.at[pl.ds(po_r, C * D)], sem_r).wait()
      pltpu.make_async_copy(src_hbm.at[pl.ds(0, C)],
                            src_v.at[pl.ds(po_e, C)], sem_s).wait()
      pltpu.make_async_copy(w_hbm.at[pl.ds(0, C)],
                            w_v.at[pl.ds(po_e, C)], sem_w).wait()

    @pl.when(k1 > k0)
    def _prologue():
      issue(k0, 0)

    def chunk_body(k, _):
      p = lax.rem(k - k0, 2)
      drain(p)

      @pl.when(k + 1 < k1)
      def _next():
        issue(k + 1, 1 - p)

      eoff = p * C

      def group_body(g, _):
        gb = eoff + g * 16
        s16 = src_v[pl.ds(gb, 16)]
        w16 = w_v[pl.ds(gb, 16)]
        j = s16 - base
        inr = (j >= 0) & (j < NPT)
        jc = jnp.where(inr, j, 0)
        rcp16 = plsc.load_gather(nbr_v, [jc + off])
        wz = jnp.where(inr, w16, 0.0) * rcp16
        j11 = jc * D
        rb = colbase + (eoff * D + g * (16 * D))
        for c in range(D):
          col = plsc.load_gather(rows_v, [rb + c])
          plsc.addupdate_scatter(acc_v, [j11 + c], col * wz)
        return 0
      lax.fori_loop(0, NG, group_body, 0)
      return 0

    lax.fori_loop(k0, k1, chunk_body, 0)

    pltpu.sync_copy(acc_v, out_hbm.at[wid])

  return seg_kernel


_seg_kernel = _build()


def kernel(poss_edge, edge_src, edge_weight, neighbours_sum):
  rows_flat = poss_edge.reshape(-1)
  src_chunks = edge_src.reshape(NCHUNK, C)
  lb = jnp.min(src_chunks, axis=1)
  ub = jnp.max(src_chunks, axis=1)
  out_raw = _seg_kernel(rows_flat, edge_src, edge_weight, neighbours_sum,
                        lb, ub)
  return out_raw[:, : NPT * D].reshape(N_NODES, D)


# speedup vs baseline: 5.0888x; 1.4691x over previous
import functools

import jax
import jax.numpy as jnp
from jax import lax
from jax.experimental import pallas as pl
from jax.experimental.pallas import tpu as pltpu
from jax.experimental.pallas import tpu_sc as plsc

N_NODES = 100000
N_EDGES = 3200000
D = 11

NC = 2
NS = 16
NW = NC * NS
NPT = N_NODES // NW
C = 2000
NCHUNK = N_EDGES // C
NG = C // 16
ACC_W = ((NPT * D + 15) // 16) * 16
NB_W = ((NPT + 7 + 15) // 16) * 16


def _build(interpret=False):
  mesh = plsc.VectorSubcoreMesh(
      core_axis_name="c", subcore_axis_name="s",
      num_cores=NC, num_subcores=NS)

  @functools.partial(
      pl.kernel,
      out_type=jax.ShapeDtypeStruct((NW, ACC_W), jnp.float32),
      mesh=mesh,
      scratch_types=[
          pltpu.VMEM((ACC_W,), jnp.float32),
          pltpu.VMEM((2 * C * D,), jnp.float32),
          pltpu.VMEM((2 * C,), jnp.int32),
          pltpu.VMEM((2 * C,), jnp.float32),
          pltpu.VMEM((NB_W,), jnp.float32),
          pltpu.VMEM((NCHUNK,), jnp.int32),
          pltpu.VMEM((NCHUNK,), jnp.int32),
          pltpu.SemaphoreType.DMA,
          pltpu.SemaphoreType.DMA,
          pltpu.SemaphoreType.DMA,
      ],
      compiler_params=pltpu.CompilerParams(needs_layout_passes=False),
      interpret=interpret,
  )
  def seg_kernel(rows_hbm, src_hbm, w_hbm, nb_hbm, lb_hbm, ub_hbm,
                 out_hbm, acc_v, rows_v, src_v, w_v, nbr_v, lb_v, ub_v,
                 sem_r, sem_s, sem_w):
    wid = lax.axis_index("c") * NS + lax.axis_index("s")
    base = wid * NPT
    limit = base + NPT

    s8 = base - lax.rem(base, 8)
    s8 = pl.multiple_of(jnp.minimum(s8, N_NODES - NB_W), 8)
    off = base - s8
    c1 = pltpu.async_copy(lb_hbm, lb_v, sem_r)
    c2 = pltpu.async_copy(ub_hbm, ub_v, sem_s)
    c3 = pltpu.async_copy(nb_hbm.at[pl.ds(s8, NB_W)], nbr_v, sem_w)
    c3.wait()
    c2.wait()
    c1.wait()

    zeros16f = jnp.zeros((16,), jnp.float32)
    iota16 = lax.iota(jnp.int32, 16)

    def init_body(i, _):
      acc_v[pl.ds(i * 16, 16)] = zeros16f
      return 0
    lax.fori_loop(0, ACC_W // 16, init_body, 0)

    def rcp_body(i, _):
      nbr_v[pl.ds(i * 16, 16)] = 1.0 / nbr_v[pl.ds(i * 16, 16)]
      return 0
    lax.fori_loop(0, NB_W // 16, rcp_body, 0)

    def cnt_body(m, carry):
      ca, cb = carry
      va = ub_v[pl.ds(m * 16, 16)]
      vb = lb_v[pl.ds(m * 16, 16)]
      ca = ca + jnp.where(va < base, 1, 0)
      cb = cb + jnp.where(vb < limit, 1, 0)
      return ca, cb
    cnt0, cnt1 = lax.fori_loop(
        0, NCHUNK // 16, cnt_body,
        (jnp.zeros((16,), jnp.int32), jnp.zeros((16,), jnp.int32)))

    def lane_sum(v):
      s = v[0]
      for l in range(1, 16):
        s = s + v[l]
      return s

    k0 = lane_sum(cnt0)
    k1 = lane_sum(cnt1)

    strip16 = iota16 * NG
    strip16d = iota16 * (NG * D)

    def issue(k, p):
      ro = pl.multiple_of(k * (C * D), 8)
      eo = pl.multiple_of(k * C, 8)
      po_r = pl.multiple_of(p * (C * D), 8)
      po_e = pl.multiple_of(p * C, 8)
      pltpu.async_copy(rows_hbm.at[pl.ds(ro, C * D)],
                       rows_v.at[pl.ds(po_r, C * D)], sem_r)
      pltpu.async_copy(src_hbm.at[pl.ds(eo, C)],
                       src_v.at[pl.ds(po_e, C)], sem_s)
      pltpu.async_copy(w_hbm.at[pl.ds(eo, C)],
                       w_v.at[pl.ds(po_e, C)], sem_w)

    def drain(p):
      po_r = pl.multiple_of(p * (C * D), 8)
      po_e = pl.multiple_of(p * C, 8)
      pltpu.make_async_copy(rows_hbm.at[pl.ds(0, C * D)],
                            rows_v.at[pl.ds(po_r, C * D)], sem_r).wait()
      pltpu.make_async_copy(src_hbm.at[pl.ds(0, C)],
                            src_v.at[pl.ds(po_e, C)], sem_s).wait()
      pltpu.make_async_copy(w_hbm.at[pl.ds(0, C)],
                            w_v.at[pl.ds(po_e, C)], sem_w).wait()

    @pl.when(k1 > k0)
    def _prologue():
      issue(k0, 0)

    def chunk_body(k, _):
      p = lax.rem(k - k0, 2)
      drain(p)

      @pl.when(k + 1 < k1)
      def _next():
        issue(k + 1, 1 - p)

      eoff = p * C

      def group_body(g, _):
        eidx = strip16 + (g + eoff)
        s16 = plsc.load_gather(src_v, [eidx])
        w16 = plsc.load_gather(w_v, [eidx])
        j = s16 - base
        inr = (j >= 0) & (j < NPT)
        jc = jnp.where(inr, j, 0)
        rcp16 = plsc.load_gather(nbr_v, [jc + off])
        wz = jnp.where(inr, w16, 0.0) * rcp16
        j11 = jc * D
        rb = strip16d + (g + eoff) * D
        for c in range(D):
          col = plsc.load_gather(rows_v, [rb + c])
          plsc.addupdate_scatter(acc_v, [j11 + c], col * wz)
        return 0
      lax.fori_loop(0, NG, group_body, 0)
      return 0

    lax.fori_loop(k0, k1, chunk_body, 0)

    pltpu.sync_copy(acc_v, out_hbm.at[wid])

  return seg_kernel


_seg_kernel = _build()


def kernel(poss_edge, edge_src, edge_weight, neighbours_sum):
  rows_flat = poss_edge.reshape(-1)
  src_chunks = edge_src.reshape(NCHUNK, C)
  lb = jnp.min(src_chunks, axis=1)
  ub = jnp.max(src_chunks, axis=1)
  out_raw = _seg_kernel(rows_flat, edge_src, edge_weight, neighbours_sum,
                        lb, ub)
  return out_raw[:, : NPT * D].reshape(N_NODES, D)


# speedup vs baseline: 5.1162x; 1.0054x over previous
import functools

import jax
import jax.numpy as jnp
from jax import lax
from jax.experimental import pallas as pl
from jax.experimental.pallas import tpu as pltpu
from jax.experimental.pallas import tpu_sc as plsc

N_NODES = 100000
N_EDGES = 3200000
D = 11

NC = 2
NS = 16
NW = NC * NS
NPT = N_NODES // NW
C = 2000
NCHUNK = N_EDGES // C
NG = C // 16
UNROLL = 5
ACC_W = ((NPT * D + 15) // 16) * 16
NB_W = ((NPT + 7 + 15) // 16) * 16


def _build(interpret=False):
  mesh = plsc.VectorSubcoreMesh(
      core_axis_name="c", subcore_axis_name="s",
      num_cores=NC, num_subcores=NS)

  @functools.partial(
      pl.kernel,
      out_type=jax.ShapeDtypeStruct((NW, ACC_W), jnp.float32),
      mesh=mesh,
      scratch_types=[
          pltpu.VMEM((ACC_W,), jnp.float32),
          pltpu.VMEM((2 * C * D,), jnp.float32),
          pltpu.VMEM((2 * C,), jnp.int32),
          pltpu.VMEM((2 * C,), jnp.float32),
          pltpu.VMEM((NB_W,), jnp.float32),
          pltpu.VMEM((NCHUNK,), jnp.int32),
          pltpu.VMEM((NCHUNK,), jnp.int32),
          pltpu.SemaphoreType.DMA,
          pltpu.SemaphoreType.DMA,
          pltpu.SemaphoreType.DMA,
      ],
      compiler_params=pltpu.CompilerParams(needs_layout_passes=False),
      interpret=interpret,
  )
  def seg_kernel(rows_hbm, src_hbm, w_hbm, nb_hbm, lb_hbm, ub_hbm,
                 out_hbm, acc_v, rows_v, src_v, w_v, nbr_v, lb_v, ub_v,
                 sem_r, sem_s, sem_w):
    wid = lax.axis_index("c") * NS + lax.axis_index("s")
    base = wid * NPT
    limit = base + NPT

    s8 = base - lax.rem(base, 8)
    s8 = pl.multiple_of(jnp.minimum(s8, N_NODES - NB_W), 8)
    off = base - s8
    c1 = pltpu.async_copy(lb_hbm, lb_v, sem_r)
    c2 = pltpu.async_copy(ub_hbm, ub_v, sem_s)
    c3 = pltpu.async_copy(nb_hbm.at[pl.ds(s8, NB_W)], nbr_v, sem_w)
    c3.wait()
    c2.wait()
    c1.wait()

    zeros16f = jnp.zeros((16,), jnp.float32)
    iota16 = lax.iota(jnp.int32, 16)

    def init_body(i, _):
      acc_v[pl.ds(i * 16, 16)] = zeros16f
      return 0
    lax.fori_loop(0, ACC_W // 16, init_body, 0)

    def rcp_body(i, _):
      nbr_v[pl.ds(i * 16, 16)] = 1.0 / nbr_v[pl.ds(i * 16, 16)]
      return 0
    lax.fori_loop(0, NB_W // 16, rcp_body, 0)

    def cnt_body(m, carry):
      ca, cb = carry
      va = ub_v[pl.ds(m * 16, 16)]
      vb = lb_v[pl.ds(m * 16, 16)]
      ca = ca + jnp.where(va < base, 1, 0)
      cb = cb + jnp.where(vb < limit, 1, 0)
      return ca, cb
    cnt0, cnt1 = lax.fori_loop(
        0, NCHUNK // 16, cnt_body,
        (jnp.zeros((16,), jnp.int32), jnp.zeros((16,), jnp.int32)))

    def lane_sum(v):
      s = v[0]
      for l in range(1, 16):
        s = s + v[l]
      return s

    k0 = lane_sum(cnt0)
    k1 = lane_sum(cnt1)

    strip16 = iota16 * NG
    strip16d = iota16 * (NG * D)

    def issue(k, p):
      ro = pl.multiple_of(k * (C * D), 8)
      eo = pl.multiple_of(k * C, 8)
      po_r = pl.multiple_of(p * (C * D), 8)
      po_e = pl.multiple_of(p * C, 8)
      pltpu.async_copy(rows_hbm.at[pl.ds(ro, C * D)],
                       rows_v.at[pl.ds(po_r, C * D)], sem_r)
      pltpu.async_copy(src_hbm.at[pl.ds(eo, C)],
                       src_v.at[pl.ds(po_e, C)], sem_s)
      pltpu.async_copy(w_hbm.at[pl.ds(eo, C)],
                       w_v.at[pl.ds(po_e, C)], sem_w)

    def drain(p):
      po_r = pl.multiple_of(p * (C * D), 8)
      po_e = pl.multiple_of(p * C, 8)
      pltpu.make_async_copy(rows_hbm.at[pl.ds(0, C * D)],
                            rows_v.at[pl.ds(po_r, C * D)], sem_r).wait()
      pltpu.make_async_copy(src_hbm.at[pl.ds(0, C)],
                            src_v.at[pl.ds(po_e, C)], sem_s).wait()
      pltpu.make_async_copy(w_hbm.at[pl.ds(0, C)],
                            w_v.at[pl.ds(po_e, C)], sem_w).wait()

    @pl.when(k1 > k0)
    def _prologue():
      issue(k0, 0)

    def chunk_body(k, _):
      p = lax.rem(k - k0, 2)
      drain(p)

      @pl.when(k + 1 < k1)
      def _next():
        issue(k + 1, 1 - p)

      eoff = p * C

      def group_body(gg, _):
        for u in range(UNROLL):
          g = gg * UNROLL + u
          eidx = strip16 + (g + eoff)
          s16 = plsc.load_gather(src_v, [eidx])
          w16 = plsc.load_gather(w_v, [eidx])
          j = s16 - base
          inr = (j >= 0) & (j < NPT)
          jc = jnp.where(inr, j, 0)
          rcp16 = plsc.load_gather(nbr_v, [jc + off])
          wz = jnp.where(inr, w16, 0.0) * rcp16
          j11 = jc * D
          rb = strip16d + (g + eoff) * D
          for c in range(D):
            col = plsc.load_gather(rows_v, [rb + c])
            plsc.addupdate_scatter(acc_v, [j11 + c], col * wz)
        return 0
      lax.fori_loop(0, NG // UNROLL, group_body, 0)
      return 0

    lax.fori_loop(k0, k1, chunk_body, 0)

    pltpu.sync_copy(acc_v, out_hbm.at[wid])

  return seg_kernel


_seg_kernel = _build()


def kernel(poss_edge, edge_src, edge_weight, neighbours_sum):
  rows_flat = poss_edge.reshape(-1)
  src_chunks = edge_src.reshape(NCHUNK, C)
  lb = jnp.min(src_chunks, axis=1)
  ub = jnp.max(src_chunks, axis=1)
  out_raw = _seg_kernel(rows_flat, edge_src, edge_weight, neighbours_sum,
                        lb, ub)
  return out_raw[:, : NPT * D].reshape(N_NODES, D)


# speedup vs baseline: 7.1187x; 1.3914x over previous
import functools

import jax
import jax.numpy as jnp
from jax import lax
from jax.experimental import pallas as pl
from jax.experimental.pallas import tpu as pltpu
from jax.experimental.pallas import tpu_sc as plsc

N_NODES = 100000
N_EDGES = 3200000
D = 11

NC = 2
NS = 16
NW = NC * NS
NPT = N_NODES // NW
C = 2560
NCHUNK = N_EDGES // C
NCH_PAD = ((NCHUNK + 15) // 16) * 16
NG = C // 16
UNROLL = 5
CD = C * D
ACC_W = ((NPT * D + 15) // 16) * 16
NB_W = ((NPT + 7 + 15) // 16) * 16
SENTINEL = 1 << 30


def _build(interpret=False):
  mesh = plsc.VectorSubcoreMesh(
      core_axis_name="c", subcore_axis_name="s",
      num_cores=NC, num_subcores=NS)

  @functools.partial(
      pl.kernel,
      out_type=jax.ShapeDtypeStruct((NW, ACC_W), jnp.float32),
      mesh=mesh,
      scratch_types=[
          pltpu.VMEM((ACC_W,), jnp.float32),
          pltpu.VMEM((2 * D, C), jnp.float32),
          pltpu.VMEM((2 * C,), jnp.int32),
          pltpu.VMEM((2 * C,), jnp.float32),
          pltpu.VMEM((NB_W,), jnp.float32),
          pltpu.VMEM((NCH_PAD,), jnp.int32),
          pltpu.VMEM((NCH_PAD,), jnp.int32),
          pltpu.SemaphoreType.DMA,
          pltpu.SemaphoreType.DMA,
          pltpu.SemaphoreType.DMA,
      ],
      compiler_params=pltpu.CompilerParams(needs_layout_passes=False),
      interpret=interpret,
  )
  def seg_kernel(rows_hbm, src_hbm, w_hbm, nb_hbm, lb_hbm, ub_hbm,
                 out_hbm, acc_v, rows_v, src_v, w_v, nbr_v, lb_v, ub_v,
                 sem_r, sem_s, sem_w):
    wid = lax.axis_index("c") * NS + lax.axis_index("s")
    base = wid * NPT
    limit = base + NPT

    s8 = base - lax.rem(base, 8)
    s8 = pl.multiple_of(jnp.minimum(s8, N_NODES - NB_W), 8)
    off = base - s8
    c1 = pltpu.async_copy(lb_hbm, lb_v, sem_r)
    c2 = pltpu.async_copy(ub_hbm, ub_v, sem_s)
    c3 = pltpu.async_copy(nb_hbm.at[pl.ds(s8, NB_W)], nbr_v, sem_w)
    c3.wait()
    c2.wait()
    c1.wait()

    zeros16f = jnp.zeros((16,), jnp.float32)
    iota16 = lax.iota(jnp.int32, 16)

    def init_body(i, _):
      acc_v[pl.ds(i * 16, 16)] = zeros16f
      return 0
    lax.fori_loop(0, ACC_W // 16, init_body, 0)

    def rcp_body(i, _):
      nbr_v[pl.ds(i * 16, 16)] = 1.0 / nbr_v[pl.ds(i * 16, 16)]
      return 0
    lax.fori_loop(0, NB_W // 16, rcp_body, 0)

    def cnt_body(m, carry):
      ca, cb = carry
      va = ub_v[pl.ds(m * 16, 16)]
      vb = lb_v[pl.ds(m * 16, 16)]
      ca = ca + jnp.where(va < base, 1, 0)
      cb = cb + jnp.where(vb < limit, 1, 0)
      return ca, cb
    cnt0, cnt1 = lax.fori_loop(
        0, NCH_PAD // 16, cnt_body,
        (jnp.zeros((16,), jnp.int32), jnp.zeros((16,), jnp.int32)))

    def lane_sum(v):
      s = v[0]
      for l in range(1, 16):
        s = s + v[l]
      return s

    k0 = lane_sum(cnt0)
    k1 = lane_sum(cnt1)

    strip16 = iota16 * NG

    def issue(k, p):
      eo = pl.multiple_of(k * C, 8)
      po_e = pl.multiple_of(p * C, 8)
      for c in range(D):
        pltpu.async_copy(rows_hbm.at[pl.ds(c, 1), pl.ds(eo, C)],
                         rows_v.at[pl.ds(p * D + c, 1), :], sem_r)
      pltpu.async_copy(src_hbm.at[pl.ds(eo, C)],
                       src_v.at[pl.ds(po_e, C)], sem_s)
      pltpu.async_copy(w_hbm.at[pl.ds(eo, C)],
                       w_v.at[pl.ds(po_e, C)], sem_w)

    def drain(p):
      po_e = pl.multiple_of(p * C, 8)
      for c in range(D):
        pltpu.make_async_copy(rows_hbm.at[pl.ds(0, 1), pl.ds(0, C)],
                              rows_v.at[pl.ds(p * D + c, 1), :], sem_r).wait()
      pltpu.make_async_copy(src_hbm.at[pl.ds(0, C)],
                            src_v.at[pl.ds(po_e, C)], sem_s).wait()
      pltpu.make_async_copy(w_hbm.at[pl.ds(0, C)],
                            w_v.at[pl.ds(po_e, C)], sem_w).wait()

    @pl.when(k1 > k0)
    def _prologue():
      issue(k0, 0)

    def chunk_body(k, _):
      p = lax.rem(k - k0, 2)
      drain(p)

      @pl.when(k + 1 < k1)
      def _next():
        issue(k + 1, 1 - p)

      eoff = p * C
      rsplat = [jnp.full((16,), 0, jnp.int32) + (p * D + c) for c in range(D)]

      def group_body(gg, _):
        for u in range(UNROLL):
          g = gg * UNROLL + u
          eidx = strip16 + (g + eoff)
          s16 = plsc.load_gather(src_v, [eidx])
          w16 = plsc.load_gather(w_v, [eidx])
          j = s16 - base
          inr = (j >= 0) & (j < NPT)
          jc = jnp.where(inr, j, 0)
          rcp16 = plsc.load_gather(nbr_v, [jc + off])
          wz = jnp.where(inr, w16, 0.0) * rcp16
          j11 = jc * D
          rb = strip16 + g
          for c in range(D):
            col = plsc.load_gather(rows_v, [rsplat[c], rb])
            plsc.addupdate_scatter(acc_v, [j11 + c], col * wz)
        return 0
      lax.fori_loop(0, NG // UNROLL, group_body, 0)
      return 0

    lax.fori_loop(k0, k1, chunk_body, 0)

    pltpu.sync_copy(acc_v, out_hbm.at[wid])

  return seg_kernel


_seg_kernel = _build()


def kernel(poss_edge, edge_src, edge_weight, neighbours_sum):
  rows_t = poss_edge.T
  src_chunks = edge_src.reshape(NCHUNK, C)
  lb = jnp.pad(jnp.min(src_chunks, axis=1), (0, NCH_PAD - NCHUNK),
               constant_values=SENTINEL)
  ub = jnp.pad(jnp.max(src_chunks, axis=1), (0, NCH_PAD - NCHUNK),
               constant_values=SENTINEL)
  out_raw = _seg_kernel(rows_t, edge_src, edge_weight, neighbours_sum,
                        lb, ub)
  return out_raw[:, : NPT * D].reshape(N_NODES, D)


# speedup vs baseline: 15.8156x; 2.2217x over previous
import functools

import jax
import jax.numpy as jnp
from jax import lax
from jax.experimental import pallas as pl
from jax.experimental.pallas import tpu as pltpu
from jax.experimental.pallas import tpu_sc as plsc

N_NODES = 100000
N_EDGES = 3200000
D = 11

NC = 2
NS = 16
NW = NC * NS
NPT = N_NODES // NW
C = 1280
NCHUNK = N_EDGES // C
NCH_PAD = ((NCHUNK + 15) // 16) * 16
NG = C // 16
UNROLL = 5
CD = C * D
ACC_W = ((NPT * D + 15) // 16) * 16
NB_W = ((NPT + 7 + 15) // 16) * 16
SENTINEL = 1 << 30


def _build(interpret=False):
  mesh = plsc.VectorSubcoreMesh(
      core_axis_name="c", subcore_axis_name="s",
      num_cores=NC, num_subcores=NS)

  @functools.partial(
      pl.kernel,
      out_type=jax.ShapeDtypeStruct((NW, ACC_W), jnp.float32),
      mesh=mesh,
      scratch_types=[
          pltpu.VMEM((ACC_W,), jnp.float32),
          pltpu.VMEM((32, C), jnp.float32),
          pltpu.VMEM((2 * C,), jnp.int32),
          pltpu.VMEM((2 * C,), jnp.float32),
          pltpu.VMEM((NB_W,), jnp.float32),
          pltpu.VMEM((NCH_PAD,), jnp.int32),
          pltpu.VMEM((NCH_PAD,), jnp.int32),
          pltpu.SemaphoreType.DMA,
          pltpu.SemaphoreType.DMA,
          pltpu.SemaphoreType.DMA,
      ],
      compiler_params=pltpu.CompilerParams(needs_layout_passes=False),
      interpret=interpret,
  )
  def seg_kernel(rows_hbm, src_hbm, w_hbm, nb_hbm, lb_hbm, ub_hbm,
                 out_hbm, acc_v, rows_v, src_v, w_v, nbr_v, lb_v, ub_v,
                 sem_r, sem_s, sem_w):
    wid = lax.axis_index("c") * NS + lax.axis_index("s")
    base = wid * NPT
    limit = base + NPT

    s8 = base - lax.rem(base, 8)
    s8 = pl.multiple_of(jnp.minimum(s8, N_NODES - NB_W), 8)
    off = base - s8
    c1 = pltpu.async_copy(lb_hbm, lb_v, sem_r)
    c2 = pltpu.async_copy(ub_hbm, ub_v, sem_s)
    c3 = pltpu.async_copy(nb_hbm.at[pl.ds(s8, NB_W)], nbr_v, sem_w)
    c3.wait()
    c2.wait()
    c1.wait()

    zeros16f = jnp.zeros((16,), jnp.float32)
    iota16 = lax.iota(jnp.int32, 16)

    def init_body(i, _):
      acc_v[pl.ds(i * 16, 16)] = zeros16f
      return 0
    lax.fori_loop(0, ACC_W // 16, init_body, 0)

    def rcp_body(i, _):
      nbr_v[pl.ds(i * 16, 16)] = 1.0 / nbr_v[pl.ds(i * 16, 16)]
      return 0
    lax.fori_loop(0, NB_W // 16, rcp_body, 0)

    def cnt_body(m, carry):
      ca, cb = carry
      va = ub_v[pl.ds(m * 16, 16)]
      vb = lb_v[pl.ds(m * 16, 16)]
      ca = ca + jnp.where(va < base, 1, 0)
      cb = cb + jnp.where(vb < limit, 1, 0)
      return ca, cb
    cnt0, cnt1 = lax.fori_loop(
        0, NCH_PAD // 16, cnt_body,
        (jnp.zeros((16,), jnp.int32), jnp.zeros((16,), jnp.int32)))

    def lane_sum(v):
      s = v[0]
      for l in range(1, 16):
        s = s + v[l]
      return s

    k0 = lane_sum(cnt0)
    k1 = lane_sum(cnt1)

    strip16 = iota16 * NG

    def issue(k, p):
      eo = pl.multiple_of(k * C, 8)
      po_e = pl.multiple_of(p * C, 8)
      po16 = pl.multiple_of(p * 16, 8)
      pltpu.async_copy(rows_hbm.at[pl.ds(0, 8), pl.ds(eo, C)],
                       rows_v.at[pl.ds(po16, 8), :], sem_r)
      for c in range(8, D):
        pltpu.async_copy(rows_hbm.at[pl.ds(c, 1), pl.ds(eo, C)],
                         rows_v.at[pl.ds(p * 16 + c, 1), :], sem_r)
      pltpu.async_copy(src_hbm.at[pl.ds(eo, C)],
                       src_v.at[pl.ds(po_e, C)], sem_s)
      pltpu.async_copy(w_hbm.at[pl.ds(eo, C)],
                       w_v.at[pl.ds(po_e, C)], sem_w)

    def drain(p):
      po_e = pl.multiple_of(p * C, 8)
      po16 = pl.multiple_of(p * 16, 8)
      pltpu.make_async_copy(rows_hbm.at[pl.ds(0, 8), pl.ds(0, C)],
                            rows_v.at[pl.ds(po16, 8), :], sem_r).wait()
      for c in range(8, D):
        pltpu.make_async_copy(rows_hbm.at[pl.ds(0, 1), pl.ds(0, C)],
                              rows_v.at[pl.ds(p * 16 + c, 1), :], sem_r).wait()
      pltpu.make_async_copy(src_hbm.at[pl.ds(0, C)],
                            src_v.at[pl.ds(po_e, C)], sem_s).wait()
      pltpu.make_async_copy(w_hbm.at[pl.ds(0, C)],
                            w_v.at[pl.ds(po_e, C)], sem_w).wait()

    @pl.when(k1 > k0)
    def _prologue():
      issue(k0, 0)

    def chunk_body(k, _):
      p = lax.rem(k - k0, 2)
      drain(p)

      @pl.when(k + 1 < k1)
      def _next():
        issue(k + 1, 1 - p)

      eoff = p * C
      rsplat = [jnp.full((16,), 0, jnp.int32) + (p * 16 + c) for c in range(D)]

      def group_body(gg, _):
        for u in range(UNROLL):
          g = gg * UNROLL + u
          eidx = strip16 + (g + eoff)
          s16 = plsc.load_gather(src_v, [eidx])
          w16 = plsc.load_gather(w_v, [eidx])
          j = s16 - base
          inr = (j >= 0) & (j < NPT)
          jc = jnp.where(inr, j, 0)
          rcp16 = plsc.load_gather(nbr_v, [jc + off])
          wz = jnp.where(inr, w16, 0.0) * rcp16
          j11 = jc * D
          rb = strip16 + g
          cols = [plsc.load_gather(rows_v, [rsplat[c], rb]) for c in range(D)]
          vals = [cols[c] * wz for c in range(D)]
          for c in range(D):
            plsc.addupdate_scatter(acc_v, [j11 + c], vals[c])
        return 0
      lax.fori_loop(0, NG // UNROLL, group_body, 0)
      return 0

    lax.fori_loop(k0, k1, chunk_body, 0)

    pltpu.sync_copy(acc_v, out_hbm.at[wid])

  return seg_kernel


_seg_kernel = _build()


def kernel(poss_edge, edge_src, edge_weight, neighbours_sum):
  rows_t = poss_edge.T
  src_chunks = edge_src.reshape(NCHUNK, C)
  lb = jnp.pad(jnp.min(src_chunks, axis=1), (0, NCH_PAD - NCHUNK),
               constant_values=SENTINEL)
  ub = jnp.pad(jnp.max(src_chunks, axis=1), (0, NCH_PAD - NCHUNK),
               constant_values=SENTINEL)
  out_raw = _seg_kernel(rows_t, edge_src, edge_weight, neighbours_sum,
                        lb, ub)
  return out_raw[:, : NPT * D].reshape(N_NODES, D)
